# Initial kernel scaffold; baseline (speedup 1.0000x reference)
import jax
import jax.numpy as jnp
from jax.experimental import pallas as pl


def kernel(target_input, context, neg, emb):
    raise NotImplementedError("write your pallas kernel here")


# speedup vs baseline: 7.0435x; 7.0435x over previous
import functools

import jax
import jax.numpy as jnp
from jax import lax
from jax.experimental import pallas as pl
from jax.experimental.pallas import tpu as pltpu
from jax.experimental.pallas import tpu_sc as plsc

B_ = 16384
DIM_ = 64
NEG_ = 20
NC_ = 2
NS_ = 16
NW_ = NC_ * NS_
BPW_ = B_ // NW_
C_ = 32
NCHUNK_ = BPW_ // C_
NIDX_ROWS_ = C_ * NEG_ // 128


def _sc_dots(tgt, ctx, negf, emb):
    mesh = plsc.VectorSubcoreMesh(core_axis_name="c", subcore_axis_name="s")

    @functools.partial(
        pl.kernel,
        mesh=mesh,
        compiler_params=pltpu.CompilerParams(
            needs_layout_passes=False, use_tc_tiling_on_sc=False),
        out_type=(jax.ShapeDtypeStruct((B_,), jnp.float32),
                  jax.ShapeDtypeStruct((B_,), jnp.float32)),
        scratch_types=[
            pltpu.VMEM((NCHUNK_, C_), jnp.int32),
            pltpu.VMEM((NCHUNK_, C_), jnp.int32),
            pltpu.VMEM((NCHUNK_ * NIDX_ROWS_, 128), jnp.int32),
            pltpu.VMEM((C_, DIM_), jnp.float32),
            pltpu.VMEM((C_, DIM_), jnp.float32),
            pltpu.VMEM((C_ * NEG_, DIM_), jnp.float32),
            pltpu.VMEM((C_, DIM_), jnp.float32),
            pltpu.VMEM((C_,), jnp.float32),
            pltpu.VMEM((C_,), jnp.float32),
            pltpu.SemaphoreType.DMA,
        ],
    )
    def run(tgt_hbm, ctx_hbm, negf_hbm, emb_hbm, dpos_hbm, dneg_hbm,
            tidx, cidx, nidx, vrows, urows, nrows, nsumv, dposv, dnegv, sem):
        wid = lax.axis_index("s") * NC_ + lax.axis_index("c")
        base = wid * BPW_
        pltpu.sync_copy(tgt_hbm.at[wid], tidx)
        pltpu.sync_copy(ctx_hbm.at[wid], cidx)
        pltpu.sync_copy(negf_hbm.at[wid], nidx)

        def chunk_body(i, carry):
            start = base + i * C_
            cps = [pltpu.async_copy(emb_hbm.at[tidx.at[i]], vrows, sem),
                   pltpu.async_copy(emb_hbm.at[cidx.at[i]], urows, sem)]
            for j in range(NIDX_ROWS_):
                cps.append(pltpu.async_copy(
                    emb_hbm.at[nidx.at[i * NIDX_ROWS_ + j]],
                    nrows.at[pl.ds(j * 128, 128)], sem))
            for cp in cps:
                cp.wait()
            for b in range(C_):
                sq = [nrows[b * NEG_, pl.ds(q * 16, 16)] for q in range(4)]
                for n in range(1, NEG_):
                    for q in range(4):
                        sq[q] = sq[q] + nrows[b * NEG_ + n, pl.ds(q * 16, 16)]
                for q in range(4):
                    nsumv[b, pl.ds(q * 16, 16)] = sq[q]
            lanes = lax.iota(jnp.int32, 16)
            for g in range(C_ // 16):
                bidx = lanes + g * 16
                pacc = jnp.zeros((16,), jnp.float32)
                nacc = jnp.zeros((16,), jnp.float32)
                for d in range(DIM_):
                    dd = jnp.full((16,), d, jnp.int32)
                    vT = plsc.load_gather(vrows, [bidx, dd])
                    uT = plsc.load_gather(urows, [bidx, dd])
                    nT = plsc.load_gather(nsumv, [bidx, dd])
                    pacc = pacc + vT * uT
                    nacc = nacc + vT * nT
                dposv[pl.ds(g * 16, 16)] = pacc
                dnegv[pl.ds(g * 16, 16)] = nacc
            pltpu.sync_copy(dposv, dpos_hbm.at[pl.ds(start, C_)])
            pltpu.sync_copy(dnegv, dneg_hbm.at[pl.ds(start, C_)])
            return carry

        lax.fori_loop(0, NCHUNK_, chunk_body, 0)

    return run(tgt, ctx, negf, emb)


def _loss_body(dp_ref, dn_ref, o_ref):
    dp = dp_ref[...]
    dn = dn_ref[...]
    ls = (jnp.minimum(dp, 0.0) - jnp.log1p(jnp.exp(-jnp.abs(dp)))
          + jnp.minimum(-dn, 0.0) - jnp.log1p(jnp.exp(-jnp.abs(dn))))
    o_ref[0, 0] = -jnp.sum(ls) / B_


def _loss_tc(dpos, dneg):
    return pl.pallas_call(
        _loss_body,
        out_shape=jax.ShapeDtypeStruct((1, 1), jnp.float32),
        out_specs=pl.BlockSpec(memory_space=pltpu.SMEM),
    )(dpos, dneg)


def kernel(target_input, context, neg, emb):
    tgt = target_input.astype(jnp.int32).reshape(NW_, NCHUNK_, C_)
    ctx = context.astype(jnp.int32).reshape(NW_, NCHUNK_, C_)
    negf = neg.astype(jnp.int32).reshape(NW_, NCHUNK_ * NIDX_ROWS_, 128)
    dpos, dneg = _sc_dots(tgt, ctx, negf, emb)
    out = _loss_tc(dpos.reshape(128, 128), dneg.reshape(128, 128))
    return out[0, 0]
